# baseline (speedup 1.0000x reference)
import functools

import jax
import jax.numpy as jnp
from jax import lax
from jax.experimental import pallas as pl
from jax.experimental.pallas import tpu as pltpu
from jax.experimental.pallas import tpu_sc as plsc

S = 2048
D = 768
E = 8
H = 3072
K = 2
M = 256
NT = K * S // M + E
R = NT * M
HC = 3
HH = H // HC
SUB = 2


def _dg(a, b, dims):
    return lax.dot_general(a, b, (dims, ((), ())),
                           preferred_element_type=jnp.float32)


def _router_body(x_ref, rw_ref, probs_ref, usage_ref, lb_ref, z_ref,
                 w0_ref, w1_ref, d0_ref, d1_ref, te_ref, nu_ref):
    x = x_ref[...]
    rw = rw_ref[...]
    logits = _dg(x, rw, ((1,), (1,)))

    lane = lax.broadcasted_iota(jnp.int32, (S, E), 1)
    m = jnp.max(logits, axis=1, keepdims=True)
    ex = jnp.exp(logits - m)
    se = jnp.sum(ex, axis=1, keepdims=True)
    probs = ex / se
    probs_ref[...] = probs
    usage = jnp.sum(probs, axis=0, keepdims=True) * (1.0 / S)
    usage_ref[...] = usage
    lb_ref[...] = jnp.reshape(jnp.sum(usage * usage) * E, (1, 1))
    lse = m + jnp.log(se)
    z_ref[...] = jnp.reshape(jnp.sum(lse * lse) * (1.0 / S), (1, 1))

    i1 = jnp.min(jnp.where(logits == m, lane, E), axis=1, keepdims=True)
    masked = jnp.where(lane == i1, -1e30, logits)
    l2 = jnp.max(masked, axis=1, keepdims=True)
    i2 = jnp.min(jnp.where(masked == l2, lane, E), axis=1, keepdims=True)
    e2 = jnp.exp(l2 - m)
    w0_ref[...] = 1.0 / (1.0 + e2)
    w1_ref[...] = e2 / (1.0 + e2)

    oh0 = (lane == i1).astype(jnp.float32)
    oh1 = (lane == i2).astype(jnp.float32)
    ohs = oh0 + oh1
    cc_in = ohs
    k = 1
    while k < S:
        shifted = jnp.concatenate(
            [jnp.zeros((k, E), jnp.float32), cc_in[:S - k]], axis=0)
        cc_in = cc_in + shifted
        k *= 2
    cc_ex = cc_in - ohs
    ones_col = jnp.ones((S, 1), jnp.float32)
    counts = _dg(ohs, ones_col, ((0,), (0,)))
    padded = jnp.ceil(counts * (1.0 / M)) * M
    er = lax.broadcasted_iota(jnp.int32, (E, E), 0)
    ec = lax.broadcasted_iota(jnp.int32, (E, E), 1)
    tri_e = (er > ec).astype(jnp.float32)
    start = _dg(tri_e, padded, ((1,), (0,)))
    s0 = _dg(oh0, start, ((1,), (0,)))
    s1 = _dg(oh1, start, ((1,), (0,)))
    r0 = jnp.sum(cc_ex * oh0, axis=1, keepdims=True)
    r1 = jnp.sum((cc_ex + oh0) * oh1, axis=1, keepdims=True)
    d0_ref[...] = (s0 + r0).astype(jnp.int32)
    d1_ref[...] = (s1 + r1).astype(jnp.int32)

    end = start + padded
    tpos = lax.broadcasted_iota(jnp.int32, (1, 128), 1).astype(jnp.float32) * M
    owner = jnp.sum((end <= tpos).astype(jnp.int32), axis=0, keepdims=True)
    te_ref[...] = jnp.minimum(owner, E - 1)
    nu_ref[...] = jnp.reshape(jnp.sum(padded) * (1.0 / M), (1, 1)).astype(jnp.int32)


def _router_dispatch(x2, router_W):
    f32 = jnp.float32
    i32 = jnp.int32
    outs = pl.pallas_call(
        _router_body,
        out_shape=[
            jax.ShapeDtypeStruct((S, E), f32),
            jax.ShapeDtypeStruct((1, E), f32),
            jax.ShapeDtypeStruct((1, 1), f32),
            jax.ShapeDtypeStruct((1, 1), f32),
            jax.ShapeDtypeStruct((S, 1), f32),
            jax.ShapeDtypeStruct((S, 1), f32),
            jax.ShapeDtypeStruct((S, 1), i32),
            jax.ShapeDtypeStruct((S, 1), i32),
            jax.ShapeDtypeStruct((1, 128), i32),
            jax.ShapeDtypeStruct((1, 1), i32),
        ],
    )(x2, router_W)
    return outs


def _sc_dispatch(x2, d0, d1, ch, ncores):
    mesh = plsc.VectorSubcoreMesh(core_axis_name="c", subcore_axis_name="s")

    @functools.partial(
        pl.kernel, mesh=mesh,
        out_type=jax.ShapeDtypeStruct((R, D), jnp.float32),
        scratch_types=[
            pltpu.VMEM((ch,), jnp.int32),
            pltpu.VMEM((ch,), jnp.int32),
            pltpu.VMEM((ch, D), jnp.float32),
            pltpu.SemaphoreType.DMA,
            pltpu.SemaphoreType.DMA,
            pltpu.SemaphoreType.DMA,
        ],
    )
    def k(x_hbm, d0_hbm, d1_hbm, out_hbm, i0_v, i1_v, rows_v, s0, s1, s2):
        w = lax.axis_index("s") * ncores + lax.axis_index("c")
        base = w * ch
        c0 = pltpu.async_copy(d0_hbm.at[pl.ds(base, ch)], i0_v, s0)
        c1 = pltpu.async_copy(d1_hbm.at[pl.ds(base, ch)], i1_v, s1)
        c2 = pltpu.async_copy(x_hbm.at[pl.ds(base, ch)], rows_v, s2)
        c0.wait()
        c1.wait()
        c2.wait()
        t0 = pltpu.async_copy(rows_v, out_hbm.at[i0_v], s0)
        t1 = pltpu.async_copy(rows_v, out_hbm.at[i1_v], s1)
        t0.wait()
        t1.wait()

    return k(x2, d0, d1)


def _ffn_body(te_ref, nu_ref, x_ref, w1_ref, w3_ref, w2_ref, y_ref, acc_ref):
    h = pl.program_id(0)
    t = pl.program_id(1)

    @pl.when(t < nu_ref[0])
    def _():
        xb = x_ref[...]
        g = _dg(xb, w1_ref[0], ((1,), (1,)))
        u = _dg(xb, w3_ref[0], ((1,), (1,)))
        g = g * (1.0 / (1.0 + jnp.exp(-g)))
        part = _dg(g * u, w2_ref[0], ((1,), (1,)))

        if HC == 1:
            y_ref[...] = part
        else:
            @pl.when(h == 0)
            def _():
                acc_ref[pl.ds(t * M, M), :] = part

            @pl.when(h == HC - 1)
            def _():
                y_ref[...] = acc_ref[pl.ds(t * M, M), :] + part


def _grouped_ffn(te, nu, xs, W1, W2, W3):
    grid_spec = pltpu.PrefetchScalarGridSpec(
        num_scalar_prefetch=2,
        grid=(HC, NT),
        in_specs=[
            pl.BlockSpec((M, D), lambda h, t, te, nu: (t, 0)),
            pl.BlockSpec((1, HH, D), lambda h, t, te, nu: (te[t], h, 0)),
            pl.BlockSpec((1, HH, D), lambda h, t, te, nu: (te[t], h, 0)),
            pl.BlockSpec((1, D, HH), lambda h, t, te, nu: (te[t], 0, h)),
        ],
        out_specs=pl.BlockSpec(
            (M, D),
            lambda h, t, te, nu: (jnp.where(h == HC - 1, t, 0), 0)),
        scratch_shapes=[pltpu.VMEM((8 if HC == 1 else R, D), jnp.float32)],
    )
    return pl.pallas_call(
        _ffn_body,
        grid_spec=grid_spec,
        out_shape=jax.ShapeDtypeStruct((R, D), jnp.float32),
    )(te, nu, xs, W1, W3, W2)


def _sc_gather(y, d0, d1, ch, ncores):
    mesh = plsc.VectorSubcoreMesh(core_axis_name="c", subcore_axis_name="s")

    @functools.partial(
        pl.kernel, mesh=mesh,
        out_type=(jax.ShapeDtypeStruct((S, D), jnp.float32),
                  jax.ShapeDtypeStruct((S, D), jnp.float32)),
        scratch_types=[
            pltpu.VMEM((ch,), jnp.int32),
            pltpu.VMEM((ch,), jnp.int32),
            pltpu.VMEM((ch, D), jnp.float32),
            pltpu.VMEM((ch, D), jnp.float32),
            pltpu.SemaphoreType.DMA,
            pltpu.SemaphoreType.DMA,
        ],
    )
    def k(y_hbm, d0_hbm, d1_hbm, o0_hbm, o1_hbm,
          i0_v, i1_v, rows0_v, rows1_v, s0, s1):
        w = lax.axis_index("s") * ncores + lax.axis_index("c")
        base = w * ch
        c0 = pltpu.async_copy(d0_hbm.at[pl.ds(base, ch)], i0_v, s0)
        c1 = pltpu.async_copy(d1_hbm.at[pl.ds(base, ch)], i1_v, s1)
        c0.wait()
        g0 = pltpu.async_copy(y_hbm.at[i0_v], rows0_v, s0)
        c1.wait()
        g1 = pltpu.async_copy(y_hbm.at[i1_v], rows1_v, s1)
        g0.wait()
        o0 = pltpu.async_copy(rows0_v, o0_hbm.at[pl.ds(base, ch)], s0)
        g1.wait()
        o1 = pltpu.async_copy(rows1_v, o1_hbm.at[pl.ds(base, ch)], s1)
        o0.wait()
        o1.wait()

    return k(y, d0, d1)


def _combine_body(y0_ref, y1_ref, w0_ref, w1_ref, o_ref):
    o_ref[...] = w0_ref[...] * y0_ref[...] + w1_ref[...] * y1_ref[...]


def _combine(yp0, yp1, w0, w1):
    return pl.pallas_call(
        _combine_body,
        out_shape=jax.ShapeDtypeStruct((S, D), jnp.float32),
    )(yp0, yp1, w0, w1)


def kernel(x, router_W, W1, W2, W3):
    x2 = x.reshape(S, D)
    (probs, usage, lb, z, w0, w1, d0, d1, te, nu) = _router_dispatch(
        x2, router_W)
    d0f = d0.reshape(S)
    d1f = d1.reshape(S)

    info = plsc.get_sparse_core_info()
    nw = info.num_cores * info.num_subcores
    ch = S // nw

    xs = _sc_dispatch(x2, d0f, d1f, ch, info.num_cores)
    y = _grouped_ffn(te[0, :NT], nu.reshape(1), xs, W1, W2, W3)
    yp0, yp1 = _sc_gather(y, d0f, d1f, ch, info.num_cores)
    out = _combine(yp0, yp1, w0, w1)

    return (out.reshape(1, S, D), lb.reshape(()), z.reshape(()),
            usage.reshape(E), probs.reshape(1, S, E))


# speedup vs baseline: 1.1818x; 1.1818x over previous
import functools

import jax
import jax.numpy as jnp
from jax import lax
from jax.experimental import pallas as pl
from jax.experimental.pallas import tpu as pltpu
from jax.experimental.pallas import tpu_sc as plsc

S = 2048
D = 768
E = 8
H = 3072
K = 2
M = 256
NT = K * S // M + E
R = NT * M
HC = 1
HH = H // HC
SUB = 2


def _dg(a, b, dims):
    return lax.dot_general(a, b, (dims, ((), ())),
                           preferred_element_type=jnp.float32)


def _router_body(x_ref, rw_ref, probs_ref, usage_ref, lb_ref, z_ref,
                 w0_ref, w1_ref, d0_ref, d1_ref, te_ref, nu_ref):
    x = x_ref[...]
    rw = rw_ref[...]
    logits = _dg(x, rw, ((1,), (1,)))

    lane = lax.broadcasted_iota(jnp.int32, (S, E), 1)
    m = jnp.max(logits, axis=1, keepdims=True)
    ex = jnp.exp(logits - m)
    se = jnp.sum(ex, axis=1, keepdims=True)
    probs = ex / se
    probs_ref[...] = probs
    usage = jnp.sum(probs, axis=0, keepdims=True) * (1.0 / S)
    usage_ref[...] = usage
    lb_ref[...] = jnp.reshape(jnp.sum(usage * usage) * E, (1, 1))
    lse = m + jnp.log(se)
    z_ref[...] = jnp.reshape(jnp.sum(lse * lse) * (1.0 / S), (1, 1))

    i1 = jnp.min(jnp.where(logits == m, lane, E), axis=1, keepdims=True)
    masked = jnp.where(lane == i1, -1e30, logits)
    l2 = jnp.max(masked, axis=1, keepdims=True)
    i2 = jnp.min(jnp.where(masked == l2, lane, E), axis=1, keepdims=True)
    e2 = jnp.exp(l2 - m)
    w0_ref[...] = 1.0 / (1.0 + e2)
    w1_ref[...] = e2 / (1.0 + e2)

    oh0 = (lane == i1).astype(jnp.float32)
    oh1 = (lane == i2).astype(jnp.float32)
    ohs = oh0 + oh1
    cc_in = ohs
    k = 1
    while k < S:
        shifted = jnp.concatenate(
            [jnp.zeros((k, E), jnp.float32), cc_in[:S - k]], axis=0)
        cc_in = cc_in + shifted
        k *= 2
    cc_ex = cc_in - ohs
    ones_col = jnp.ones((S, 1), jnp.float32)
    counts = _dg(ohs, ones_col, ((0,), (0,)))
    padded = jnp.ceil(counts * (1.0 / M)) * M
    er = lax.broadcasted_iota(jnp.int32, (E, E), 0)
    ec = lax.broadcasted_iota(jnp.int32, (E, E), 1)
    tri_e = (er > ec).astype(jnp.float32)
    start = _dg(tri_e, padded, ((1,), (0,)))
    s0 = _dg(oh0, start, ((1,), (0,)))
    s1 = _dg(oh1, start, ((1,), (0,)))
    r0 = jnp.sum(cc_ex * oh0, axis=1, keepdims=True)
    r1 = jnp.sum((cc_ex + oh0) * oh1, axis=1, keepdims=True)
    d0_ref[...] = (s0 + r0).astype(jnp.int32)
    d1_ref[...] = (s1 + r1).astype(jnp.int32)

    end = start + padded
    tpos = lax.broadcasted_iota(jnp.int32, (1, 128), 1).astype(jnp.float32) * M
    owner = jnp.sum((end <= tpos).astype(jnp.int32), axis=0, keepdims=True)
    te_ref[...] = jnp.minimum(owner, E - 1)
    nu_ref[...] = jnp.reshape(jnp.sum(padded) * (1.0 / M), (1, 1)).astype(jnp.int32)


def _router_dispatch(x2, router_W):
    f32 = jnp.float32
    i32 = jnp.int32
    outs = pl.pallas_call(
        _router_body,
        out_shape=[
            jax.ShapeDtypeStruct((S, E), f32),
            jax.ShapeDtypeStruct((1, E), f32),
            jax.ShapeDtypeStruct((1, 1), f32),
            jax.ShapeDtypeStruct((1, 1), f32),
            jax.ShapeDtypeStruct((S, 1), f32),
            jax.ShapeDtypeStruct((S, 1), f32),
            jax.ShapeDtypeStruct((S, 1), i32),
            jax.ShapeDtypeStruct((S, 1), i32),
            jax.ShapeDtypeStruct((1, 128), i32),
            jax.ShapeDtypeStruct((1, 1), i32),
        ],
    )(x2, router_W)
    return outs


def _sc_dispatch(x2, d0, d1, ch, ncores):
    mesh = plsc.VectorSubcoreMesh(core_axis_name="c", subcore_axis_name="s")

    @functools.partial(
        pl.kernel, mesh=mesh,
        out_type=jax.ShapeDtypeStruct((R, D), jnp.float32),
        scratch_types=[
            pltpu.VMEM((ch,), jnp.int32),
            pltpu.VMEM((ch,), jnp.int32),
            pltpu.VMEM((ch, D), jnp.float32),
            pltpu.SemaphoreType.DMA,
            pltpu.SemaphoreType.DMA,
            pltpu.SemaphoreType.DMA,
        ],
    )
    def k(x_hbm, d0_hbm, d1_hbm, out_hbm, i0_v, i1_v, rows_v, s0, s1, s2):
        w = lax.axis_index("s") * ncores + lax.axis_index("c")
        base = w * ch
        c0 = pltpu.async_copy(d0_hbm.at[pl.ds(base, ch)], i0_v, s0)
        c1 = pltpu.async_copy(d1_hbm.at[pl.ds(base, ch)], i1_v, s1)
        c2 = pltpu.async_copy(x_hbm.at[pl.ds(base, ch)], rows_v, s2)
        c0.wait()
        c1.wait()
        c2.wait()
        t0 = pltpu.async_copy(rows_v, out_hbm.at[i0_v], s0)
        t1 = pltpu.async_copy(rows_v, out_hbm.at[i1_v], s1)
        t0.wait()
        t1.wait()

    return k(x2, d0, d1)


def _ffn_body(te_ref, nu_ref, x_ref, w1_ref, w3_ref, w2_ref, y_ref, acc_ref):
    h = pl.program_id(0)
    t = pl.program_id(1)

    @pl.when(t < nu_ref[0])
    def _():
        xb = x_ref[...]
        g = _dg(xb, w1_ref[0], ((1,), (1,)))
        u = _dg(xb, w3_ref[0], ((1,), (1,)))
        g = g * (1.0 / (1.0 + jnp.exp(-g)))
        part = _dg(g * u, w2_ref[0], ((1,), (1,)))

        if HC == 1:
            y_ref[...] = part
        else:
            @pl.when(h == 0)
            def _():
                acc_ref[pl.ds(t * M, M), :] = part

            if HC > 2:
                @pl.when(jnp.logical_and(h > 0, h < HC - 1))
                def _():
                    acc_ref[pl.ds(t * M, M), :] += part

            @pl.when(h == HC - 1)
            def _():
                y_ref[...] = acc_ref[pl.ds(t * M, M), :] + part


def _grouped_ffn(te, nu, xs, W1, W2, W3):
    grid_spec = pltpu.PrefetchScalarGridSpec(
        num_scalar_prefetch=2,
        grid=(HC, NT),
        in_specs=[
            pl.BlockSpec((M, D), lambda h, t, te, nu: (t, 0)),
            pl.BlockSpec((1, HH, D), lambda h, t, te, nu: (te[t], h, 0)),
            pl.BlockSpec((1, HH, D), lambda h, t, te, nu: (te[t], h, 0)),
            pl.BlockSpec((1, D, HH), lambda h, t, te, nu: (te[t], 0, h)),
        ],
        out_specs=pl.BlockSpec(
            (M, D),
            lambda h, t, te, nu: (jnp.where(h == HC - 1, t, 0), 0)),
        scratch_shapes=[pltpu.VMEM((8 if HC == 1 else R, D), jnp.float32)],
    )
    return pl.pallas_call(
        _ffn_body,
        grid_spec=grid_spec,
        out_shape=jax.ShapeDtypeStruct((R, D), jnp.float32),
        compiler_params=pltpu.CompilerParams(
            vmem_limit_bytes=100 * 1024 * 1024),
    )(te, nu, xs, W1, W3, W2)


def _sc_gather(y, d0, d1, ch, ncores):
    mesh = plsc.VectorSubcoreMesh(core_axis_name="c", subcore_axis_name="s")

    @functools.partial(
        pl.kernel, mesh=mesh,
        out_type=(jax.ShapeDtypeStruct((S, D), jnp.float32),
                  jax.ShapeDtypeStruct((S, D), jnp.float32)),
        scratch_types=[
            pltpu.VMEM((ch,), jnp.int32),
            pltpu.VMEM((ch,), jnp.int32),
            pltpu.VMEM((ch, D), jnp.float32),
            pltpu.VMEM((ch, D), jnp.float32),
            pltpu.SemaphoreType.DMA,
            pltpu.SemaphoreType.DMA,
        ],
    )
    def k(y_hbm, d0_hbm, d1_hbm, o0_hbm, o1_hbm,
          i0_v, i1_v, rows0_v, rows1_v, s0, s1):
        w = lax.axis_index("s") * ncores + lax.axis_index("c")
        base = w * ch
        c0 = pltpu.async_copy(d0_hbm.at[pl.ds(base, ch)], i0_v, s0)
        c1 = pltpu.async_copy(d1_hbm.at[pl.ds(base, ch)], i1_v, s1)
        c0.wait()
        g0 = pltpu.async_copy(y_hbm.at[i0_v], rows0_v, s0)
        c1.wait()
        g1 = pltpu.async_copy(y_hbm.at[i1_v], rows1_v, s1)
        g0.wait()
        o0 = pltpu.async_copy(rows0_v, o0_hbm.at[pl.ds(base, ch)], s0)
        g1.wait()
        o1 = pltpu.async_copy(rows1_v, o1_hbm.at[pl.ds(base, ch)], s1)
        o0.wait()
        o1.wait()

    return k(y, d0, d1)


def _combine_body(y0_ref, y1_ref, w0_ref, w1_ref, o_ref):
    o_ref[...] = w0_ref[...] * y0_ref[...] + w1_ref[...] * y1_ref[...]


def _combine(yp0, yp1, w0, w1):
    return pl.pallas_call(
        _combine_body,
        out_shape=jax.ShapeDtypeStruct((S, D), jnp.float32),
    )(yp0, yp1, w0, w1)


def kernel(x, router_W, W1, W2, W3):
    x2 = x.reshape(S, D)
    (probs, usage, lb, z, w0, w1, d0, d1, te, nu) = _router_dispatch(
        x2, router_W)
    d0f = d0.reshape(S)
    d1f = d1.reshape(S)

    info = plsc.get_sparse_core_info()
    nw = info.num_cores * info.num_subcores
    ch = S // nw

    xs = _sc_dispatch(x2, d0f, d1f, ch, info.num_cores)
    y = _grouped_ffn(te[0, :NT], nu.reshape(1), xs, W1, W2, W3)
    yp0, yp1 = _sc_gather(y, d0f, d1f, ch, info.num_cores)
    out = _combine(yp0, yp1, w0, w1)

    return (out.reshape(1, S, D), lb.reshape(()), z.reshape(()),
            usage.reshape(E), probs.reshape(1, S, E))


# speedup vs baseline: 1.2560x; 1.0627x over previous
import functools

import jax
import jax.numpy as jnp
from jax import lax
from jax.experimental import pallas as pl
from jax.experimental.pallas import tpu as pltpu
from jax.experimental.pallas import tpu_sc as plsc

S = 2048
D = 768
E = 8
H = 3072
K = 2
M = 256
NT = K * S // M + E
R = NT * M
HC = 1
HH = H // HC
NTE = 128
NSLOT = 2


def _dg(a, b, dims):
    return lax.dot_general(a, b, (dims, ((), ())),
                           preferred_element_type=jnp.float32)


def _router_body(x_ref, rw_ref, probs_ref, usage_ref, lb_ref, z_ref,
                 w0_ref, w1_ref, d0_ref, d1_ref, meta_ref):
    x = x_ref[...]
    rw = rw_ref[...]
    logits = _dg(x, rw, ((1,), (1,)))

    lane = lax.broadcasted_iota(jnp.int32, (S, E), 1)
    m = jnp.max(logits, axis=1, keepdims=True)
    ex = jnp.exp(logits - m)
    se = jnp.sum(ex, axis=1, keepdims=True)
    probs = ex / se
    probs_ref[...] = probs
    usage = jnp.sum(probs, axis=0, keepdims=True) * (1.0 / S)
    usage_ref[...] = usage
    lb_ref[...] = jnp.reshape(jnp.sum(usage * usage) * E, (1, 1))
    lse = m + jnp.log(se)
    z_ref[...] = jnp.reshape(jnp.sum(lse * lse) * (1.0 / S), (1, 1))

    i1 = jnp.min(jnp.where(logits == m, lane, E), axis=1, keepdims=True)
    masked = jnp.where(lane == i1, -1e30, logits)
    l2 = jnp.max(masked, axis=1, keepdims=True)
    i2 = jnp.min(jnp.where(masked == l2, lane, E), axis=1, keepdims=True)
    e2 = jnp.exp(l2 - m)
    w0_ref[...] = 1.0 / (1.0 + e2)
    w1_ref[...] = e2 / (1.0 + e2)

    oh0 = (lane == i1).astype(jnp.float32)
    oh1 = (lane == i2).astype(jnp.float32)
    ohs = oh0 + oh1
    cc_in = ohs
    k = 1
    while k < S:
        shifted = jnp.concatenate(
            [jnp.zeros((k, E), jnp.float32), cc_in[:S - k]], axis=0)
        cc_in = cc_in + shifted
        k *= 2
    cc_ex = cc_in - ohs
    ones_col = jnp.ones((S, 1), jnp.float32)
    counts = _dg(ohs, ones_col, ((0,), (0,)))
    padded = jnp.ceil(counts * (1.0 / M)) * M
    er = lax.broadcasted_iota(jnp.int32, (E, E), 0)
    ec = lax.broadcasted_iota(jnp.int32, (E, E), 1)
    tri_e = (er > ec).astype(jnp.float32)
    start = _dg(tri_e, padded, ((1,), (0,)))
    s0 = _dg(oh0, start, ((1,), (0,)))
    s1 = _dg(oh1, start, ((1,), (0,)))
    r0 = jnp.sum(cc_ex * oh0, axis=1, keepdims=True)
    r1 = jnp.sum((cc_ex + oh0) * oh1, axis=1, keepdims=True)
    d0_ref[...] = (s0 + r0).astype(jnp.int32)
    d1_ref[...] = (s1 + r1).astype(jnp.int32)

    i8 = (er == ec).astype(jnp.float32)
    end_row = _dg(start + padded, i8, ((0,), (0,)))
    tcol = lax.broadcasted_iota(jnp.int32, (NTE, 1), 0)
    tpos = tcol.astype(jnp.float32) * M
    owner = jnp.sum((end_row <= tpos).astype(jnp.int32), axis=1, keepdims=True)
    te_col = jnp.minimum(owner, E - 1)
    te_prev = jnp.concatenate([te_col[:1], te_col[:-1]], axis=0)
    eo_col = (te_col != te_prev).astype(jnp.int32)
    k = 1
    while k < NTE:
        eo_col = eo_col + jnp.concatenate(
            [jnp.zeros((k, 1), jnp.int32), eo_col[:NTE - k]], axis=0)
        k *= 2
    i128 = (lax.broadcasted_iota(jnp.int32, (NTE, NTE), 0) ==
            lax.broadcasted_iota(jnp.int32, (NTE, NTE), 1)).astype(jnp.float32)
    eo_row = _dg(eo_col.astype(jnp.float32), i128, ((0,), (0,)))
    te_row = _dg(te_col.astype(jnp.float32), i128, ((0,), (0,)))
    mask = tcol.astype(jnp.float32) == eo_row
    fte_col = jnp.min(jnp.where(mask, te_row, 99.0), axis=1,
                      keepdims=True).astype(jnp.int32)
    nu = (jnp.sum(padded) * (1.0 / M)).astype(jnp.int32)
    used = tcol < nu
    ne = jnp.max(jnp.where(used, eo_col, 0)) + 1
    col3 = (jnp.where(tcol == 0, nu, 0) + jnp.where(tcol == 1, ne, 0))
    meta_ref[...] = jnp.concatenate([te_col, eo_col, fte_col, col3], axis=1)


def _router_dispatch(x2, router_W):
    f32 = jnp.float32
    i32 = jnp.int32
    outs = pl.pallas_call(
        _router_body,
        out_shape=[
            jax.ShapeDtypeStruct((S, E), f32),
            jax.ShapeDtypeStruct((1, E), f32),
            jax.ShapeDtypeStruct((1, 1), f32),
            jax.ShapeDtypeStruct((1, 1), f32),
            jax.ShapeDtypeStruct((S, 1), f32),
            jax.ShapeDtypeStruct((S, 1), f32),
            jax.ShapeDtypeStruct((S, 1), i32),
            jax.ShapeDtypeStruct((S, 1), i32),
            jax.ShapeDtypeStruct((NTE, 4), i32),
        ],
    )(x2, router_W)
    return outs


def _sc_dispatch(x2, d0, d1, ch, ncores):
    mesh = plsc.VectorSubcoreMesh(core_axis_name="c", subcore_axis_name="s")

    @functools.partial(
        pl.kernel, mesh=mesh,
        out_type=jax.ShapeDtypeStruct((R, D), jnp.float32),
        scratch_types=[
            pltpu.VMEM((ch,), jnp.int32),
            pltpu.VMEM((ch,), jnp.int32),
            pltpu.VMEM((ch, D), jnp.float32),
            pltpu.SemaphoreType.DMA,
            pltpu.SemaphoreType.DMA,
            pltpu.SemaphoreType.DMA,
        ],
    )
    def k(x_hbm, d0_hbm, d1_hbm, out_hbm, i0_v, i1_v, rows_v, s0, s1, s2):
        w = lax.axis_index("s") * ncores + lax.axis_index("c")
        base = w * ch
        c0 = pltpu.async_copy(d0_hbm.at[pl.ds(base, ch)], i0_v, s0)
        c1 = pltpu.async_copy(d1_hbm.at[pl.ds(base, ch)], i1_v, s1)
        c2 = pltpu.async_copy(x_hbm.at[pl.ds(base, ch)], rows_v, s2)
        c0.wait()
        c1.wait()
        c2.wait()
        t0 = pltpu.async_copy(rows_v, out_hbm.at[i0_v], s0)
        t1 = pltpu.async_copy(rows_v, out_hbm.at[i1_v], s1)
        t0.wait()
        t1.wait()

    return k(x2, d0, d1)


def _ffn_body(meta_ref, x_ref, w1_hbm, w3_hbm, w2_hbm, y_ref,
              w1b, w3b, w2b, sems):
    t = pl.program_id(0)
    nu = meta_ref[0, 3]
    ne = meta_ref[1, 3]

    def issue(ordinal, slot):
        e = meta_ref[ordinal, 2]
        pltpu.make_async_copy(w1_hbm.at[e], w1b.at[slot],
                              sems.at[slot, 0]).start()
        pltpu.make_async_copy(w3_hbm.at[e], w3b.at[slot],
                              sems.at[slot, 1]).start()
        pltpu.make_async_copy(w2_hbm.at[e], w2b.at[slot],
                              sems.at[slot, 2]).start()

    def wait(slot):
        pltpu.make_async_copy(w1_hbm.at[0], w1b.at[slot],
                              sems.at[slot, 0]).wait()
        pltpu.make_async_copy(w3_hbm.at[0], w3b.at[slot],
                              sems.at[slot, 1]).wait()
        pltpu.make_async_copy(w2_hbm.at[0], w2b.at[slot],
                              sems.at[slot, 2]).wait()

    @pl.when(t == 0)
    def _():
        issue(0, 0)
        issue(1, 1)

    @pl.when(t < nu)
    def _():
        eo = meta_ref[t, 1]
        slot = lax.rem(eo, NSLOT)
        is_first = jnp.logical_or(
            t == 0, meta_ref[jnp.maximum(t - 1, 0), 0] != meta_ref[t, 0])

        @pl.when(is_first)
        def _():
            wait(slot)

        xb = x_ref[...]
        g = _dg(xb, w1b[slot], ((1,), (1,)))
        u = _dg(xb, w3b[slot], ((1,), (1,)))
        g = g * (1.0 / (1.0 + jnp.exp(-g)))
        y_ref[...] = _dg(g * u, w2b[slot], ((1,), (1,)))

        is_last = meta_ref[t + 1, 0] != meta_ref[t, 0]

        @pl.when(jnp.logical_and(is_last, eo + 2 < ne))
        def _():
            issue(eo + 2, slot)


def _grouped_ffn(meta, xs, W1, W2, W3):
    grid_spec = pltpu.PrefetchScalarGridSpec(
        num_scalar_prefetch=1,
        grid=(NT,),
        in_specs=[
            pl.BlockSpec((M, D), lambda t, meta: (t, 0)),
            pl.BlockSpec(memory_space=pltpu.MemorySpace.HBM),
            pl.BlockSpec(memory_space=pltpu.MemorySpace.HBM),
            pl.BlockSpec(memory_space=pltpu.MemorySpace.HBM),
        ],
        out_specs=pl.BlockSpec((M, D), lambda t, meta: (t, 0)),
        scratch_shapes=[
            pltpu.VMEM((NSLOT, H, D), jnp.float32),
            pltpu.VMEM((NSLOT, H, D), jnp.float32),
            pltpu.VMEM((NSLOT, D, H), jnp.float32),
            pltpu.SemaphoreType.DMA((NSLOT, 3)),
        ],
    )
    return pl.pallas_call(
        _ffn_body,
        grid_spec=grid_spec,
        out_shape=jax.ShapeDtypeStruct((R, D), jnp.float32),
        compiler_params=pltpu.CompilerParams(
            vmem_limit_bytes=112 * 1024 * 1024),
    )(meta, xs, W1, W3, W2)


def _sc_gather(y, d0, d1, ch, ncores):
    mesh = plsc.VectorSubcoreMesh(core_axis_name="c", subcore_axis_name="s")

    @functools.partial(
        pl.kernel, mesh=mesh,
        out_type=(jax.ShapeDtypeStruct((S, D), jnp.float32),
                  jax.ShapeDtypeStruct((S, D), jnp.float32)),
        scratch_types=[
            pltpu.VMEM((ch,), jnp.int32),
            pltpu.VMEM((ch,), jnp.int32),
            pltpu.VMEM((ch, D), jnp.float32),
            pltpu.VMEM((ch, D), jnp.float32),
            pltpu.SemaphoreType.DMA,
            pltpu.SemaphoreType.DMA,
        ],
    )
    def k(y_hbm, d0_hbm, d1_hbm, o0_hbm, o1_hbm,
          i0_v, i1_v, rows0_v, rows1_v, s0, s1):
        w = lax.axis_index("s") * ncores + lax.axis_index("c")
        base = w * ch
        c0 = pltpu.async_copy(d0_hbm.at[pl.ds(base, ch)], i0_v, s0)
        c1 = pltpu.async_copy(d1_hbm.at[pl.ds(base, ch)], i1_v, s1)
        c0.wait()
        g0 = pltpu.async_copy(y_hbm.at[i0_v], rows0_v, s0)
        c1.wait()
        g1 = pltpu.async_copy(y_hbm.at[i1_v], rows1_v, s1)
        g0.wait()
        o0 = pltpu.async_copy(rows0_v, o0_hbm.at[pl.ds(base, ch)], s0)
        g1.wait()
        o1 = pltpu.async_copy(rows1_v, o1_hbm.at[pl.ds(base, ch)], s1)
        o0.wait()
        o1.wait()

    return k(y, d0, d1)


def _combine_body(y0_ref, y1_ref, w0_ref, w1_ref, o_ref):
    o_ref[...] = w0_ref[...] * y0_ref[...] + w1_ref[...] * y1_ref[...]


def _combine(yp0, yp1, w0, w1):
    return pl.pallas_call(
        _combine_body,
        out_shape=jax.ShapeDtypeStruct((S, D), jnp.float32),
    )(yp0, yp1, w0, w1)


def kernel(x, router_W, W1, W2, W3):
    x2 = x.reshape(S, D)
    (probs, usage, lb, z, w0, w1, d0, d1, meta) = _router_dispatch(
        x2, router_W)
    d0f = d0.reshape(S)
    d1f = d1.reshape(S)

    info = plsc.get_sparse_core_info()
    nw = info.num_cores * info.num_subcores
    ch = S // nw

    xs = _sc_dispatch(x2, d0f, d1f, ch, info.num_cores)
    y = _grouped_ffn(meta, xs, W1, W2, W3)
    yp0, yp1 = _sc_gather(y, d0f, d1f, ch, info.num_cores)
    out = _combine(yp0, yp1, w0, w1)

    return (out.reshape(1, S, D), lb.reshape(()), z.reshape(()),
            usage.reshape(E), probs.reshape(1, S, E))


# speedup vs baseline: 1.3777x; 1.0969x over previous
import functools

import jax
import jax.numpy as jnp
from jax import lax
from jax.experimental import pallas as pl
from jax.experimental.pallas import tpu as pltpu
from jax.experimental.pallas import tpu_sc as plsc

S = 2048
D = 768
E = 8
H = 3072
K = 2
M = 256
NT = K * S // M + E
R = NT * M
HC = 1
HH = H // HC
NTE = 128
NSLOT = 2


def _dg(a, b, dims):
    return lax.dot_general(a, b, (dims, ((), ())),
                           preferred_element_type=jnp.float32)


def _router_body(x_ref, rw_ref, probs_ref, usage_ref, lb_ref, z_ref,
                 w0_ref, w1_ref, d0_ref, d1_ref, meta_ref):
    x = x_ref[...]
    rw = rw_ref[...]
    logits = _dg(x, rw, ((1,), (1,)))

    lane = lax.broadcasted_iota(jnp.int32, (S, E), 1)
    m = jnp.max(logits, axis=1, keepdims=True)
    ex = jnp.exp(logits - m)
    se = jnp.sum(ex, axis=1, keepdims=True)
    probs = ex / se
    probs_ref[...] = probs
    usage = jnp.sum(probs, axis=0, keepdims=True) * (1.0 / S)
    usage_ref[...] = usage
    lb_ref[...] = jnp.reshape(jnp.sum(usage * usage) * E, (1, 1))
    lse = m + jnp.log(se)
    z_ref[...] = jnp.reshape(jnp.sum(lse * lse) * (1.0 / S), (1, 1))

    i1 = jnp.min(jnp.where(logits == m, lane, E), axis=1, keepdims=True)
    masked = jnp.where(lane == i1, -1e30, logits)
    l2 = jnp.max(masked, axis=1, keepdims=True)
    i2 = jnp.min(jnp.where(masked == l2, lane, E), axis=1, keepdims=True)
    e2 = jnp.exp(l2 - m)
    w0_ref[...] = 1.0 / (1.0 + e2)
    w1_ref[...] = e2 / (1.0 + e2)

    oh0 = (lane == i1).astype(jnp.float32)
    oh1 = (lane == i2).astype(jnp.float32)
    ohs = oh0 + oh1
    cc_in = ohs
    k = 1
    while k < S:
        shifted = jnp.concatenate(
            [jnp.zeros((k, E), jnp.float32), cc_in[:S - k]], axis=0)
        cc_in = cc_in + shifted
        k *= 2
    cc_ex = cc_in - ohs
    ones_col = jnp.ones((S, 1), jnp.float32)
    counts = _dg(ohs, ones_col, ((0,), (0,)))
    padded = jnp.ceil(counts * (1.0 / M)) * M
    er = lax.broadcasted_iota(jnp.int32, (E, E), 0)
    ec = lax.broadcasted_iota(jnp.int32, (E, E), 1)
    tri_e = (er > ec).astype(jnp.float32)
    start = _dg(tri_e, padded, ((1,), (0,)))
    s0 = _dg(oh0, start, ((1,), (0,)))
    s1 = _dg(oh1, start, ((1,), (0,)))
    r0 = jnp.sum(cc_ex * oh0, axis=1, keepdims=True)
    r1 = jnp.sum((cc_ex + oh0) * oh1, axis=1, keepdims=True)
    d0_ref[...] = (s0 + r0).astype(jnp.int32)
    d1_ref[...] = (s1 + r1).astype(jnp.int32)

    i8 = (er == ec).astype(jnp.float32)
    end_row = _dg(start + padded, i8, ((0,), (0,)))
    tcol = lax.broadcasted_iota(jnp.int32, (NTE, 1), 0)
    tpos = tcol.astype(jnp.float32) * M
    owner = jnp.sum((end_row <= tpos).astype(jnp.int32), axis=1, keepdims=True)
    te_col = jnp.minimum(owner, E - 1)
    te_prev = jnp.concatenate([te_col[:1], te_col[:-1]], axis=0)
    eo_col = (te_col != te_prev).astype(jnp.int32)
    k = 1
    while k < NTE:
        eo_col = eo_col + jnp.concatenate(
            [jnp.zeros((k, 1), jnp.int32), eo_col[:NTE - k]], axis=0)
        k *= 2
    i128 = (lax.broadcasted_iota(jnp.int32, (NTE, NTE), 0) ==
            lax.broadcasted_iota(jnp.int32, (NTE, NTE), 1)).astype(jnp.float32)
    eo_row = _dg(eo_col.astype(jnp.float32), i128, ((0,), (0,)))
    te_row = _dg(te_col.astype(jnp.float32), i128, ((0,), (0,)))
    mask = tcol.astype(jnp.float32) == eo_row
    fte_col = jnp.min(jnp.where(mask, te_row, 99.0), axis=1,
                      keepdims=True).astype(jnp.int32)
    nu = (jnp.sum(padded) * (1.0 / M)).astype(jnp.int32)
    used = tcol < nu
    ne = jnp.max(jnp.where(used, eo_col, 0)) + 1
    col3 = (jnp.where(tcol == 0, nu, 0) + jnp.where(tcol == 1, ne, 0))
    meta_ref[...] = jnp.concatenate([te_col, eo_col, fte_col, col3], axis=1)


def _router_dispatch(x2, router_W):
    f32 = jnp.float32
    i32 = jnp.int32
    outs = pl.pallas_call(
        _router_body,
        out_shape=[
            jax.ShapeDtypeStruct((S, E), f32),
            jax.ShapeDtypeStruct((1, E), f32),
            jax.ShapeDtypeStruct((1, 1), f32),
            jax.ShapeDtypeStruct((1, 1), f32),
            jax.ShapeDtypeStruct((S, 1), f32),
            jax.ShapeDtypeStruct((S, 1), f32),
            jax.ShapeDtypeStruct((S, 1), i32),
            jax.ShapeDtypeStruct((S, 1), i32),
            jax.ShapeDtypeStruct((NTE, 4), i32),
        ],
    )(x2, router_W)
    return outs


def _sc_dispatch(x2, d0, d1, ch, ncores):
    mesh = plsc.VectorSubcoreMesh(core_axis_name="c", subcore_axis_name="s")

    @functools.partial(
        pl.kernel, mesh=mesh,
        out_type=jax.ShapeDtypeStruct((R, D), jnp.float32),
        scratch_types=[
            pltpu.VMEM((ch,), jnp.int32),
            pltpu.VMEM((ch,), jnp.int32),
            pltpu.VMEM((ch, D), jnp.float32),
            pltpu.SemaphoreType.DMA,
            pltpu.SemaphoreType.DMA,
            pltpu.SemaphoreType.DMA,
        ],
    )
    def k(x_hbm, d0_hbm, d1_hbm, out_hbm, i0_v, i1_v, rows_v, s0, s1, s2):
        w = lax.axis_index("s") * ncores + lax.axis_index("c")
        base = w * ch
        c0 = pltpu.async_copy(d0_hbm.at[pl.ds(base, ch)], i0_v, s0)
        c1 = pltpu.async_copy(d1_hbm.at[pl.ds(base, ch)], i1_v, s1)
        c2 = pltpu.async_copy(x_hbm.at[pl.ds(base, ch)], rows_v, s2)
        c0.wait()
        c1.wait()
        c2.wait()
        t0 = pltpu.async_copy(rows_v, out_hbm.at[i0_v], s0)
        t1 = pltpu.async_copy(rows_v, out_hbm.at[i1_v], s1)
        t0.wait()
        t1.wait()

    return k(x2, d0, d1)


def _ffn_body(meta_ref, x_ref, w1_hbm, w3_hbm, w2_hbm, y_ref,
              w1b, w3b, w2b, sems):
    t = pl.program_id(0)
    nu = meta_ref[0, 3]
    ne = meta_ref[1, 3]

    def issue(ordinal, slot):
        e = meta_ref[ordinal, 2]
        pltpu.make_async_copy(w1_hbm.at[e], w1b.at[slot],
                              sems.at[slot, 0]).start()
        pltpu.make_async_copy(w3_hbm.at[e], w3b.at[slot],
                              sems.at[slot, 1]).start()
        pltpu.make_async_copy(w2_hbm.at[e], w2b.at[slot],
                              sems.at[slot, 2]).start()

    def wait(slot):
        pltpu.make_async_copy(w1_hbm.at[0], w1b.at[slot],
                              sems.at[slot, 0]).wait()
        pltpu.make_async_copy(w3_hbm.at[0], w3b.at[slot],
                              sems.at[slot, 1]).wait()
        pltpu.make_async_copy(w2_hbm.at[0], w2b.at[slot],
                              sems.at[slot, 2]).wait()

    @pl.when(t == 0)
    def _():
        issue(0, 0)

    @pl.when(t < nu)
    def _():
        eo = meta_ref[t, 1]
        slot = lax.rem(eo, NSLOT)
        is_first = jnp.logical_or(
            t == 0, meta_ref[jnp.maximum(t - 1, 0), 0] != meta_ref[t, 0])

        @pl.when(is_first)
        def _():
            wait(slot)
            @pl.when(eo + 1 < ne)
            def _():
                issue(eo + 1, lax.rem(eo + 1, NSLOT))

        xb = x_ref[...]
        g = _dg(xb, w1b[slot], ((1,), (1,)))
        u = _dg(xb, w3b[slot], ((1,), (1,)))
        g = g * (1.0 / (1.0 + jnp.exp(-g)))
        y_ref[...] = _dg(g * u, w2b[slot], ((1,), (1,)))


def _grouped_ffn(meta, xs, W1, W2, W3):
    grid_spec = pltpu.PrefetchScalarGridSpec(
        num_scalar_prefetch=1,
        grid=(NT,),
        in_specs=[
            pl.BlockSpec((M, D), lambda t, meta: (t, 0)),
            pl.BlockSpec(memory_space=pltpu.MemorySpace.HBM),
            pl.BlockSpec(memory_space=pltpu.MemorySpace.HBM),
            pl.BlockSpec(memory_space=pltpu.MemorySpace.HBM),
        ],
        out_specs=pl.BlockSpec((M, D), lambda t, meta: (t, 0)),
        scratch_shapes=[
            pltpu.VMEM((NSLOT, H, D), jnp.float32),
            pltpu.VMEM((NSLOT, H, D), jnp.float32),
            pltpu.VMEM((NSLOT, D, H), jnp.float32),
            pltpu.SemaphoreType.DMA((NSLOT, 3)),
        ],
    )
    return pl.pallas_call(
        _ffn_body,
        grid_spec=grid_spec,
        out_shape=jax.ShapeDtypeStruct((R, D), jnp.float32),
        compiler_params=pltpu.CompilerParams(
            vmem_limit_bytes=112 * 1024 * 1024),
    )(meta, xs, W1, W3, W2)


def _sc_gather(y, d0, d1, ch, ncores):
    mesh = plsc.VectorSubcoreMesh(core_axis_name="c", subcore_axis_name="s")

    @functools.partial(
        pl.kernel, mesh=mesh,
        out_type=(jax.ShapeDtypeStruct((S, D), jnp.float32),
                  jax.ShapeDtypeStruct((S, D), jnp.float32)),
        scratch_types=[
            pltpu.VMEM((ch,), jnp.int32),
            pltpu.VMEM((ch,), jnp.int32),
            pltpu.VMEM((ch, D), jnp.float32),
            pltpu.VMEM((ch, D), jnp.float32),
            pltpu.SemaphoreType.DMA,
            pltpu.SemaphoreType.DMA,
        ],
    )
    def k(y_hbm, d0_hbm, d1_hbm, o0_hbm, o1_hbm,
          i0_v, i1_v, rows0_v, rows1_v, s0, s1):
        w = lax.axis_index("s") * ncores + lax.axis_index("c")
        base = w * ch
        c0 = pltpu.async_copy(d0_hbm.at[pl.ds(base, ch)], i0_v, s0)
        c1 = pltpu.async_copy(d1_hbm.at[pl.ds(base, ch)], i1_v, s1)
        c0.wait()
        g0 = pltpu.async_copy(y_hbm.at[i0_v], rows0_v, s0)
        c1.wait()
        g1 = pltpu.async_copy(y_hbm.at[i1_v], rows1_v, s1)
        g0.wait()
        o0 = pltpu.async_copy(rows0_v, o0_hbm.at[pl.ds(base, ch)], s0)
        g1.wait()
        o1 = pltpu.async_copy(rows1_v, o1_hbm.at[pl.ds(base, ch)], s1)
        o0.wait()
        o1.wait()

    return k(y, d0, d1)


def _combine_body(y0_ref, y1_ref, w0_ref, w1_ref, o_ref):
    o_ref[...] = w0_ref[...] * y0_ref[...] + w1_ref[...] * y1_ref[...]


def _combine(yp0, yp1, w0, w1):
    return pl.pallas_call(
        _combine_body,
        out_shape=jax.ShapeDtypeStruct((S, D), jnp.float32),
    )(yp0, yp1, w0, w1)


def kernel(x, router_W, W1, W2, W3):
    x2 = x.reshape(S, D)
    (probs, usage, lb, z, w0, w1, d0, d1, meta) = _router_dispatch(
        x2, router_W)
    d0f = d0.reshape(S)
    d1f = d1.reshape(S)

    info = plsc.get_sparse_core_info()
    nw = info.num_cores * info.num_subcores
    ch = S // nw

    xs = _sc_dispatch(x2, d0f, d1f, ch, info.num_cores)
    y = _grouped_ffn(meta, xs, W1, W2, W3)
    yp0, yp1 = _sc_gather(y, d0f, d1f, ch, info.num_cores)
    out = _combine(yp0, yp1, w0, w1)

    return (out.reshape(1, S, D), lb.reshape(()), z.reshape(()),
            usage.reshape(E), probs.reshape(1, S, E))
